# baseline (speedup 1.0000x reference)
import jax
import jax.numpy as jnp
from jax import lax
from jax.experimental import pallas as pl
from jax.experimental.pallas import tpu as pltpu
from jax.experimental.pallas import tpu_sc as plsc

B = 16384
F = 128
P = 26
NM = 25
OUT_W = F + 2

NC = 2
NS = 16
NW = NC * NS
ROWS = B // NW
CHUNK = 256
NCHUNK = ROWS // CHUNK


def _sc_body(feat_hbm, par_hbm, out_hbm, feat_v, par_v, pair_v, sem_in, sem_out):
    wid = lax.axis_index("s") * NC + lax.axis_index("c")
    lane = lax.iota(jnp.int32, 16)

    for c in range(NCHUNK):
        base = wid * ROWS + c * CHUNK

        in_f = pltpu.async_copy(feat_hbm.at[pl.ds(base, CHUNK), :], feat_v, sem_in)
        in_p = pltpu.async_copy(par_hbm.at[pl.ds(base, CHUNK), :], par_v, sem_in)
        in_p.wait()
        in_f.wait()

        feat_copies = [
            pltpu.async_copy(
                feat_v, out_hbm.at[pl.ds(base, CHUNK), i, pl.ds(0, F)], sem_out
            )
            for i in range(NM)
        ]

        def row_body(r, carry):
            rr = jnp.full((16,), r, dtype=jnp.int32)
            for j in range(4):
                k = lane + 16 * j
                col = jnp.minimum(k // 2 + k % 2, P - 1)
                vals = plsc.load_gather(par_v, [rr, col])
                plsc.store_scatter(
                    pair_v,
                    [rr, jnp.minimum(k // 2, NM - 1), k % 2],
                    vals,
                    mask=k < 2 * NM,
                )
            return carry

        lax.fori_loop(0, CHUNK, row_body, 0)

        pair_copy = pltpu.async_copy(
            pair_v, out_hbm.at[pl.ds(base, CHUNK), :, pl.ds(F, 2)], sem_out
        )

        for fc in feat_copies:
            fc.wait()
        pair_copy.wait()


@jax.jit
def kernel(features, parameters):
    mesh = plsc.VectorSubcoreMesh(
        core_axis_name="c", subcore_axis_name="s", num_cores=NC, num_subcores=NS
    )
    run = pl.kernel(
        _sc_body,
        out_type=jax.ShapeDtypeStruct((B, NM, OUT_W), jnp.float32),
        mesh=mesh,
        scratch_types=[
            pltpu.VMEM((CHUNK, F), jnp.float32),
            pltpu.VMEM((CHUNK, P), jnp.float32),
            pltpu.VMEM((CHUNK, NM, 2), jnp.float32),
            pltpu.SemaphoreType.DMA,
            pltpu.SemaphoreType.DMA,
        ],
        compiler_params=pltpu.CompilerParams(
            use_tc_tiling_on_sc=False, needs_layout_passes=False
        ),
    )
    return run(features, parameters)
